# baseline (speedup 1.0000x reference)
import functools

import jax
import jax.numpy as jnp
from jax import lax
from jax.experimental import pallas as pl
from jax.experimental.pallas import tpu as pltpu
from jax.experimental.pallas import tpu_sc as plsc


def _make_gather(V, D, B):
  info = plsc.get_sparse_core_info()
  NW = info.num_cores * info.num_subcores
  assert B % (8 * NW) == 0 and D % info.num_lanes == 0
  b_per_w = B // NW
  mesh = plsc.VectorSubcoreMesh(core_axis_name="c", subcore_axis_name="s")

  @functools.partial(
      pl.kernel,
      mesh=mesh,
      compiler_params=pltpu.CompilerParams(use_tc_tiling_on_sc=False),
      out_type=jax.ShapeDtypeStruct((B, D), jnp.float32),
      scratch_types=[
          pltpu.VMEM((b_per_w,), jnp.int32),
          pltpu.VMEM((b_per_w, D), jnp.float32),
          pltpu.SemaphoreType.DMA,
      ],
  )
  def gather_kernel(table_hbm, idx_hbm, out_hbm, idx_v, rows_v, sem):
    wid = lax.axis_index("s") * info.num_cores + lax.axis_index("c")
    base = wid * b_per_w
    pltpu.sync_copy(idx_hbm.at[pl.ds(base, b_per_w)], idx_v)
    pltpu.async_copy(table_hbm.at[idx_v], rows_v, sem).wait()
    pltpu.sync_copy(rows_v, out_hbm.at[pl.ds(base, b_per_w)])

  return gather_kernel


def kernel(x, weight):
  V, D = weight.shape
  B = x.shape[0]
  return _make_gather(V, D, B)(weight, x.astype(jnp.int32))


# speedup vs baseline: 3.2512x; 3.2512x over previous
import functools

import jax
import jax.numpy as jnp
from jax import lax
from jax.experimental import pallas as pl
from jax.experimental.pallas import tpu as pltpu
from jax.experimental.pallas import tpu_sc as plsc

_LANES = 16
_RING = 4


def _make_colgather(V, D, B):
  info = plsc.get_sparse_core_info()
  NC, NS = info.num_cores, info.num_subcores
  NW = NC * NS
  assert B % (128 * NW) == 0 and D == 32
  b_per_w = B // NW
  mesh = plsc.VectorSubcoreMesh(core_axis_name="c", subcore_axis_name="s")

  @functools.partial(
      pl.kernel,
      mesh=mesh,
      compiler_params=pltpu.CompilerParams(needs_layout_passes=False),
      out_type=jax.ShapeDtypeStruct((D, B), jnp.float32),
      scratch_types=[
          pltpu.VMEM((b_per_w,), jnp.int32),
          [pltpu.VMEM((D, 128), jnp.float32) for _ in range(_RING)],
          pltpu.VMEM((D, 128), jnp.float32),
          [pltpu.SemaphoreType.DMA for _ in range(_RING)],
      ],
  )
  def gather_kernel(w_hbm, idx_hbm, out_hbm, idx_v, wins, slab, sems):
    wid = lax.axis_index("s") * NC + lax.axis_index("c")
    base = wid * b_per_w
    pltpu.sync_copy(idx_hbm.at[pl.ds(base, b_per_w)], idx_v)
    lanes = lax.iota(jnp.int32, _LANES)

    def fire(b, slot):
      c = plsc.load_gather(idx_v, [jnp.full((_LANES,), b, jnp.int32)])[0]
      c0 = pl.multiple_of((c // 128) * 128, 128)
      return pltpu.async_copy(w_hbm.at[:, pl.ds(c0, 128)], wins[slot], sems[slot])

    def drain(b, slot, copy):
      copy.wait()
      c = plsc.load_gather(idx_v, [jnp.full((_LANES,), b, jnp.int32)])[0]
      off = jnp.full((_LANES,), c % 128, jnp.int32)
      bcol = jnp.full((_LANES,), b % 128, jnp.int32)
      lo = plsc.load_gather(wins[slot], [lanes, off])
      hi = plsc.load_gather(wins[slot], [_LANES + lanes, off])
      plsc.store_scatter(slab, [lanes, bcol], lo)
      plsc.store_scatter(slab, [_LANES + lanes, bcol], hi)

    def block(k, carry):
      bstart = k * 128
      copies = [fire(bstart + s, s) for s in range(_RING)]

      def body(i, carry):
        for s in range(_RING):
          b = bstart + i * _RING + s
          drain(b, s, pltpu.make_async_copy(
              w_hbm.at[:, pl.ds(0, 128)], wins[s], sems[s]))
          nxt = b + _RING

          @pl.when(nxt < bstart + 128)
          def _():
            fire(nxt, s)

        return carry

      lax.fori_loop(0, 128 // _RING, body, 0)
      pltpu.sync_copy(slab, out_hbm.at[:, pl.ds(base + bstart, 128)])
      return carry

    lax.fori_loop(0, b_per_w // 128, block, 0)

  return gather_kernel


def kernel(x, weight):
  V, D = weight.shape
  B = x.shape[0]
  out_t = _make_colgather(V, D, B)(weight.T, x.astype(jnp.int32))
  return out_t.T


# speedup vs baseline: 4.1312x; 1.2707x over previous
import functools

import jax
import jax.numpy as jnp
from jax import lax
from jax.experimental import pallas as pl
from jax.experimental.pallas import tpu as pltpu
from jax.experimental.pallas import tpu_sc as plsc

_LANES = 16
_RING = 8


def _make_colgather(V, D, B):
  info = plsc.get_sparse_core_info()
  NC, NS = info.num_cores, info.num_subcores
  NW = NC * NS
  assert B % (128 * NW) == 0 and D == 32
  b_per_w = B // NW
  mesh = plsc.VectorSubcoreMesh(core_axis_name="c", subcore_axis_name="s")

  @functools.partial(
      pl.kernel,
      mesh=mesh,
      compiler_params=pltpu.CompilerParams(needs_layout_passes=False),
      out_type=jax.ShapeDtypeStruct((D, B), jnp.float32),
      scratch_types=[
          pltpu.VMEM((b_per_w,), jnp.int32),
          [pltpu.VMEM((D, 128), jnp.float32) for _ in range(_RING)],
          pltpu.VMEM((D, 128), jnp.float32),
          [pltpu.SemaphoreType.DMA for _ in range(_RING)],
      ],
  )
  def gather_kernel(w_hbm, idx_hbm, out_hbm, idx_v, wins, slab, sems):
    wid = lax.axis_index("s") * NC + lax.axis_index("c")
    base = wid * b_per_w
    pltpu.sync_copy(idx_hbm.at[pl.ds(base, b_per_w)], idx_v)
    lanes = lax.iota(jnp.int32, _LANES)

    def fire(b, slot):
      c = plsc.load_gather(idx_v, [jnp.full((_LANES,), b, jnp.int32)])[0]
      c0 = pl.multiple_of((c // 128) * 128, 128)
      return pltpu.async_copy(w_hbm.at[:, pl.ds(c0, 128)], wins[slot], sems[slot])

    def drain(b, slot, copy):
      copy.wait()
      c = plsc.load_gather(idx_v, [jnp.full((_LANES,), b, jnp.int32)])[0]
      off = jnp.full((_LANES,), c % 128, jnp.int32)
      bcol = jnp.full((_LANES,), b % 128, jnp.int32)
      lo = plsc.load_gather(wins[slot], [lanes, off])
      hi = plsc.load_gather(wins[slot], [_LANES + lanes, off])
      plsc.store_scatter(slab, [lanes, bcol], lo)
      plsc.store_scatter(slab, [_LANES + lanes, bcol], hi)

    def block(k, carry):
      bstart = k * 128
      copies = [fire(bstart + s, s) for s in range(_RING)]

      def body(i, carry):
        for s in range(_RING):
          b = bstart + i * _RING + s
          drain(b, s, pltpu.make_async_copy(
              w_hbm.at[:, pl.ds(0, 128)], wins[s], sems[s]))
          nxt = b + _RING

          @pl.when(nxt < bstart + 128)
          def _():
            fire(nxt, s)

        return carry

      lax.fori_loop(0, 128 // _RING, body, 0)
      pltpu.sync_copy(slab, out_hbm.at[:, pl.ds(base + bstart, 128)])
      return carry

    lax.fori_loop(0, b_per_w // 128, block, 0)

  return gather_kernel


def kernel(x, weight):
  V, D = weight.shape
  B = x.shape[0]
  out_t = _make_colgather(V, D, B)(weight.T, x.astype(jnp.int32))
  return out_t.T
